# baseline (speedup 1.0000x reference)
import functools

import jax
import jax.numpy as jnp
from jax import lax
from jax.experimental import pallas as pl
from jax.experimental.pallas import tpu as pltpu
from jax.experimental.pallas import tpu_sc as plsc

B = 4
N = 50000
D = 256
K = 128
TN = 2048
NB = 25
NPAD = NB * TN
NEG = -1e30
BIGI = 2**30
HIGHEST = lax.Precision.HIGHEST


def _scorer_body(ht_ref, wt_ref, b_ref, scorer_ref, inv_ref):
    pre = jnp.dot(ht_ref[...].astype(jnp.bfloat16),
                  wt_ref[...].astype(jnp.bfloat16),
                  preferred_element_type=jnp.float32)
    s = jnp.tanh(pre + b_ref[...])
    scorer_ref[...] = s
    ss = jnp.sum(s * s, axis=1, keepdims=True)
    inv_ref[...] = jnp.broadcast_to(lax.rsqrt(ss), (B, 128))


def _scorer(ht, w_map_t, b_map_row):
    return pl.pallas_call(
        _scorer_body,
        out_shape=[
            jax.ShapeDtypeStruct((B, D), jnp.float32),
            jax.ShapeDtypeStruct((B, 128), jnp.float32),
        ],
    )(ht, w_map_t, b_map_row)


def _score_body(z_ref, s_ref, inv_ref, raw_ref):
    zb = z_ref[0]
    sv = s_ref[0]
    r = lax.dot_general(sv.astype(jnp.bfloat16), zb.astype(jnp.bfloat16),
                        (((1,), (1,)), ((), ())),
                        preferred_element_type=jnp.float32)
    raw_ref[...] = jnp.reshape(r * inv_ref[0, 0, 0], (1, 1, 1, TN))


def _score(prev_z, scorer, inv):
    return pl.pallas_call(
        _score_body,
        grid=(B, NB),
        in_specs=[
            pl.BlockSpec((1, TN, D), lambda b, i: (b, i, 0)),
            pl.BlockSpec((1, 1, D), lambda b, i: (b, 0, 0)),
            pl.BlockSpec((1, 1, 128), lambda b, i: (b, 0, 0)),
        ],
        out_specs=pl.BlockSpec((1, 1, 1, TN), lambda b, i: (b, i, 0, 0)),
        out_shape=jax.ShapeDtypeStruct((B, NB, 1, TN), jnp.float32),
        compiler_params=pltpu.CompilerParams(
            dimension_semantics=("parallel", "arbitrary")),
    )(prev_z, scorer.reshape(B, 1, D), inv.reshape(B, 1, 128))


def _topk_body(raw_ref, ids_ref, fidx_ref, tanh_ref, ent_ref, pol_ref,
               sa_ref):
    b = pl.program_id(0)
    a0 = raw_ref[0, :, 0]
    row_i = lax.broadcasted_iota(jnp.int32, (NB, TN), 0)
    lane_i = lax.broadcasted_iota(jnp.int32, (NB, TN), 1)
    lin = row_i * TN + lane_i
    a = jnp.where(lin < N, a0, NEG)

    m0 = jnp.max(a)
    ex = jnp.exp(a - m0)
    ssum = jnp.sum(ex)
    logs = jnp.log(ssum)
    ent_val = -jnp.sum(ex * (a - m0 - logs)) / ssum
    ent_ref[...] = jnp.full((1, 1, 128), ent_val, jnp.float32)

    sa_ref[...] = a
    lane1 = lax.broadcasted_iota(jnp.int32, (1, TN), 1)
    lanek = lax.broadcasted_iota(jnp.int32, (1, 128), 1)
    iota_r = lax.broadcasted_iota(jnp.int32, (NB, 1), 0)

    def body(k, carry):
        rowmax, ids, tanhs, vsum = carry
        m = jnp.max(rowmax)
        r = jnp.min(jnp.where(rowmax == m, iota_r, BIGI))
        sl = sa_ref[pl.ds(r, 1), :]
        nvec = r * TN + lane1
        n = jnp.min(jnp.where(sl == m, nvec, BIGI))
        sl2 = jnp.where(nvec == n, NEG, sl)
        sa_ref[pl.ds(r, 1), :] = sl2
        new_rm = jnp.where(iota_r == r, jnp.max(sl2), rowmax)
        ids = jnp.where(lanek == k, n, ids)
        tanhs = jnp.where(lanek == k, jnp.tanh(m), tanhs)
        return new_rm, ids, tanhs, vsum + m

    rowmax0 = jnp.max(a, axis=1, keepdims=True)
    ids0 = jnp.zeros((1, 128), jnp.int32)
    tanhs0 = jnp.zeros((1, 128), jnp.float32)
    _, ids, tanhs, vsum = lax.fori_loop(
        0, K, body, (rowmax0, ids0, tanhs0, jnp.float32(0.0)))

    ids_ref[...] = ids.reshape(1, 1, 128)
    fidx_ref[...] = (ids + b * N).reshape(1, 1, 128)
    tanh_ref[...] = tanhs.reshape(1, 1, 128)
    pol_ref[...] = jnp.full((1, 1, 128), vsum / K - m0 - logs, jnp.float32)


def _topk(raw):
    return pl.pallas_call(
        _topk_body,
        grid=(B,),
        in_specs=[pl.BlockSpec((1, NB, 1, TN), lambda b: (b, 0, 0, 0))],
        out_specs=[
            pl.BlockSpec((1, 1, 128), lambda b: (b, 0, 0)),
            pl.BlockSpec((1, 1, 128), lambda b: (b, 0, 0)),
            pl.BlockSpec((1, 1, 128), lambda b: (b, 0, 0)),
            pl.BlockSpec((1, 1, 128), lambda b: (b, 0, 0)),
            pl.BlockSpec((1, 1, 128), lambda b: (b, 0, 0)),
        ],
        out_shape=[
            jax.ShapeDtypeStruct((B, 1, 128), jnp.int32),
            jax.ShapeDtypeStruct((B, 1, 128), jnp.int32),
            jax.ShapeDtypeStruct((B, 1, 128), jnp.float32),
            jax.ShapeDtypeStruct((B, 1, 128), jnp.float32),
            jax.ShapeDtypeStruct((B, 1, 128), jnp.float32),
        ],
        scratch_shapes=[pltpu.VMEM((NB, TN), jnp.float32)],
    )(raw)


def _sc_gather(table, fidx):
    rows = B * K
    nc, ns = 2, 16
    nw = nc * ns
    per_w = rows // nw
    mesh = plsc.VectorSubcoreMesh(core_axis_name="c", subcore_axis_name="s")

    @functools.partial(
        pl.kernel,
        out_type=jax.ShapeDtypeStruct((rows, D), jnp.float32),
        mesh=mesh,
        scratch_types=[
            pltpu.VMEM((per_w,), jnp.int32),
            pltpu.VMEM((per_w, D), jnp.float32),
            pltpu.SemaphoreType.DMA,
        ],
    )
    def gather_kernel(table_hbm, idx_hbm, out_hbm, idx_v, rows_v, sem):
        wid = lax.axis_index("s") * nc + lax.axis_index("c")
        base = wid * per_w
        pltpu.sync_copy(idx_hbm.at[pl.ds(base, per_w)], idx_v)
        pltpu.async_copy(table_hbm.at[idx_v], rows_v, sem).wait()
        pltpu.sync_copy(rows_v, out_hbm.at[pl.ds(base, per_w)])

    return gather_kernel(table, fidx)


def _gru_body(zg_ref, t_ref, q_ref, wu_ref, uu_ref, bu_ref,
              wr_ref, ur_ref, br_ref, wh_ref, uh_ref, bh_ref, out_ref):
    x = zg_ref[0]
    t_col = t_ref[0]
    q = q_ref[0]
    xt = (x * t_col).astype(jnp.bfloat16)

    def nt(w, z):
        return lax.dot_general(w.astype(jnp.bfloat16), z,
                               (((1,), (1,)), ((), ())),
                               preferred_element_type=jnp.float32)

    def nn(w, z):
        return jnp.dot(w.astype(jnp.bfloat16), z.astype(jnp.bfloat16),
                       preferred_element_type=jnp.float32)

    upd = jax.nn.sigmoid(nt(wu_ref[...], xt) + nn(uu_ref[...], q)
                         + bu_ref[...])
    rst = jax.nn.sigmoid(nt(wr_ref[...], xt) + nn(ur_ref[...], q)
                         + br_ref[...])
    hc = jnp.tanh(nt(wh_ref[...], xt) + nn(uh_ref[...], rst * q)
                  + bh_ref[...])
    out_ref[0] = (1.0 - upd) * q + upd * hc


def _gru(zg, tanh_sc, prev_q, wu, uu, bu, wr, ur, br, wh, uh, bh):
    wspec = pl.BlockSpec((K, K), lambda b: (0, 0))
    return pl.pallas_call(
        _gru_body,
        grid=(B,),
        in_specs=[
            pl.BlockSpec((1, K, K), lambda b: (b, 0, 0)),
            pl.BlockSpec((1, K, 1), lambda b: (b, 0, 0)),
            pl.BlockSpec((1, K, K), lambda b: (b, 0, 0)),
            wspec, wspec, wspec, wspec, wspec, wspec, wspec, wspec, wspec,
        ],
        out_specs=pl.BlockSpec((1, K, K), lambda b: (b, 0, 0)),
        out_shape=jax.ShapeDtypeStruct((B, K, K), jnp.float32),
    )(zg, tanh_sc, prev_q, wu, uu, bu, wr, ur, br, wh, uh, bh)


def kernel(prev_Z, prev_Q, mask, ht, W_map, b_map, W_upd, U_upd, bias_upd,
           W_rst, U_rst, bias_rst, W_htl, U_htl, bias_htl):
    scorer, inv = _scorer(ht, W_map.T, b_map.reshape(1, D))
    raw = _score(prev_Z, scorer, inv)
    ids3, fidx3, tanhs3, ent3, pol3 = _topk(raw)
    ids = ids3.reshape(B, K)
    zg = _sc_gather(prev_Z.reshape(B * N, D), fidx3.reshape(B * K))
    new_q = _gru(zg.reshape(B, K, D), tanhs3.reshape(B, K, 1), prev_Q,
                 W_upd, U_upd, bias_upd, W_rst, U_rst, bias_rst,
                 W_htl, U_htl, bias_htl)
    return new_q, pol3[:, 0, 0], scorer, ent3[:, 0, 0], ids


# speedup vs baseline: 1.0566x; 1.0566x over previous
import functools

import jax
import jax.numpy as jnp
from jax import lax
from jax.experimental import pallas as pl
from jax.experimental.pallas import tpu as pltpu
from jax.experimental.pallas import tpu_sc as plsc

B = 4
N = 50000
D = 256
K = 128
TN = 2048
NB = 25
NPAD = NB * TN
NEG = -1e30
BIGI = 2**30
HIGHEST = lax.Precision.HIGHEST


def _scorer_body(ht_ref, wt_ref, b_ref, scorer_ref, inv_ref):
    pre = jnp.dot(ht_ref[...].astype(jnp.bfloat16),
                  wt_ref[...].astype(jnp.bfloat16),
                  preferred_element_type=jnp.float32)
    s = jnp.tanh(pre + b_ref[...])
    scorer_ref[...] = s
    ss = jnp.sum(s * s, axis=1, keepdims=True)
    inv_ref[...] = jnp.broadcast_to(lax.rsqrt(ss), (B, 128))


def _scorer(ht, w_map_t, b_map_row):
    return pl.pallas_call(
        _scorer_body,
        out_shape=[
            jax.ShapeDtypeStruct((B, D), jnp.float32),
            jax.ShapeDtypeStruct((B, 128), jnp.float32),
        ],
    )(ht, w_map_t, b_map_row)


def _score_body(z_ref, s_ref, inv_ref, raw_ref):
    zb = z_ref[0]
    sv = s_ref[0]
    r = lax.dot_general(sv.astype(jnp.bfloat16), zb.astype(jnp.bfloat16),
                        (((1,), (1,)), ((), ())),
                        preferred_element_type=jnp.float32)
    raw_ref[...] = jnp.reshape(r * inv_ref[0, 0, 0], (1, 1, 1, TN))


def _score(prev_z, scorer, inv):
    return pl.pallas_call(
        _score_body,
        grid=(B, NB),
        in_specs=[
            pl.BlockSpec((1, TN, D), lambda b, i: (b, i, 0)),
            pl.BlockSpec((1, 1, D), lambda b, i: (b, 0, 0)),
            pl.BlockSpec((1, 1, 128), lambda b, i: (b, 0, 0)),
        ],
        out_specs=pl.BlockSpec((1, 1, 1, TN), lambda b, i: (b, i, 0, 0)),
        out_shape=jax.ShapeDtypeStruct((B, NB, 1, TN), jnp.float32),
        compiler_params=pltpu.CompilerParams(
            dimension_semantics=("parallel", "arbitrary")),
    )(prev_z, scorer.reshape(B, 1, D), inv.reshape(B, 1, 128))


def _topk_body(raw_ref, ids_ref, fidx_ref, tanh_ref, ent_ref, pol_ref,
               sa_ref):
    a3 = raw_ref[:, :, 0, :]
    row_i = lax.broadcasted_iota(jnp.int32, (B, NB, TN), 1)
    lane_i = lax.broadcasted_iota(jnp.int32, (B, NB, TN), 2)
    lin3 = row_i * TN + lane_i
    a3 = jnp.where(lin3 < N, a3, NEG)

    m04 = jnp.max(a3, axis=(1, 2), keepdims=True)
    ex = jnp.exp(a3 - m04)
    ssum = jnp.sum(ex, axis=(1, 2), keepdims=True)
    logs = jnp.log(ssum)
    ent4 = -jnp.sum(ex * (a3 - m04 - logs), axis=(1, 2),
                    keepdims=True) / ssum
    ent_ref[...] = jnp.broadcast_to(ent4[:, :, 0], (B, 128))

    sa_ref[...] = a3.reshape(B * NB, TN)
    lane1 = lax.broadcasted_iota(jnp.int32, (1, TN), 1)
    lanek = lax.broadcasted_iota(jnp.int32, (1, 128), 1)
    iota_r = lax.broadcasted_iota(jnp.int32, (B, NB, 1), 1)
    iota_b = lax.broadcasted_iota(jnp.int32, (B, 1), 0)

    iota_b3 = lax.broadcasted_iota(jnp.int32, (B, NB, 1), 0)
    iota_r2 = lax.broadcasted_iota(jnp.int32, (NB, 1), 0)

    def body(k, carry):
        rowmax, ids, tanhs, vsum = carry
        rm_new = rowmax
        n_col = jnp.zeros((B, 1), jnp.int32)
        m_col = jnp.zeros((B, 1), jnp.float32)
        for b in range(B):
            rmb = rowmax[b]
            mb = jnp.max(rmb)
            rb = jnp.min(jnp.where(rmb == mb, iota_r2, BIGI))
            sl = sa_ref[pl.ds(b * NB + rb, 1), :]
            nvec = rb * TN + lane1
            nb = jnp.min(jnp.where(sl == mb, nvec, BIGI))
            sl2 = jnp.where(nvec == nb, NEG, sl)
            sa_ref[pl.ds(b * NB + rb, 1), :] = sl2
            rm_new = jnp.where((iota_b3 == b) & (iota_r == rb), jnp.max(sl2),
                               rm_new)
            n_col = jnp.where(iota_b == b, nb, n_col)
            m_col = jnp.where(iota_b == b, mb, m_col)
        ids = jnp.where(lanek == k, n_col, ids)
        tanhs = jnp.where(lanek == k, jnp.tanh(m_col), tanhs)
        return rm_new, ids, tanhs, vsum + m_col

    rowmax0 = jnp.max(a3, axis=2, keepdims=True)
    ids0 = jnp.zeros((B, 128), jnp.int32)
    tanhs0 = jnp.zeros((B, 128), jnp.float32)
    _, ids, tanhs, vsum = lax.fori_loop(
        0, K, body, (rowmax0, ids0, tanhs0, jnp.zeros((B, 1), jnp.float32)))

    ids_ref[...] = ids
    fidx_ref[...] = ids + iota_b * N
    tanh_ref[...] = tanhs
    pol_ref[...] = jnp.broadcast_to(vsum / K - m04[:, :, 0] - logs[:, :, 0],
                                    (B, 128))


def _topk(raw):
    return pl.pallas_call(
        _topk_body,
        out_shape=[
            jax.ShapeDtypeStruct((B, 128), jnp.int32),
            jax.ShapeDtypeStruct((B, 128), jnp.int32),
            jax.ShapeDtypeStruct((B, 128), jnp.float32),
            jax.ShapeDtypeStruct((B, 128), jnp.float32),
            jax.ShapeDtypeStruct((B, 128), jnp.float32),
        ],
        scratch_shapes=[pltpu.VMEM((B * NB, TN), jnp.float32)],
    )(raw)


def _sc_gather(table, fidx):
    rows = B * K
    nc, ns = 2, 16
    nw = nc * ns
    per_w = rows // nw
    mesh = plsc.VectorSubcoreMesh(core_axis_name="c", subcore_axis_name="s")

    @functools.partial(
        pl.kernel,
        out_type=jax.ShapeDtypeStruct((rows, D), jnp.float32),
        mesh=mesh,
        scratch_types=[
            pltpu.VMEM((per_w,), jnp.int32),
            pltpu.VMEM((per_w, D), jnp.float32),
            pltpu.SemaphoreType.DMA,
        ],
    )
    def gather_kernel(table_hbm, idx_hbm, out_hbm, idx_v, rows_v, sem):
        wid = lax.axis_index("s") * nc + lax.axis_index("c")
        base = wid * per_w
        pltpu.sync_copy(idx_hbm.at[pl.ds(base, per_w)], idx_v)
        pltpu.async_copy(table_hbm.at[idx_v], rows_v, sem).wait()
        pltpu.sync_copy(rows_v, out_hbm.at[pl.ds(base, per_w)])

    return gather_kernel(table, fidx)


def _gru_body(zg_ref, t_ref, q_ref, wu_ref, uu_ref, bu_ref,
              wr_ref, ur_ref, br_ref, wh_ref, uh_ref, bh_ref, out_ref):
    x = zg_ref[0]
    t_col = t_ref[0]
    q = q_ref[0]
    xt = (x * t_col).astype(jnp.bfloat16)

    def nt(w, z):
        return lax.dot_general(w.astype(jnp.bfloat16), z,
                               (((1,), (1,)), ((), ())),
                               preferred_element_type=jnp.float32)

    def nn(w, z):
        return jnp.dot(w.astype(jnp.bfloat16), z.astype(jnp.bfloat16),
                       preferred_element_type=jnp.float32)

    upd = jax.nn.sigmoid(nt(wu_ref[...], xt) + nn(uu_ref[...], q)
                         + bu_ref[...])
    rst = jax.nn.sigmoid(nt(wr_ref[...], xt) + nn(ur_ref[...], q)
                         + br_ref[...])
    hc = jnp.tanh(nt(wh_ref[...], xt) + nn(uh_ref[...], rst * q)
                  + bh_ref[...])
    out_ref[0] = (1.0 - upd) * q + upd * hc


def _gru(zg, tanh_sc, prev_q, wu, uu, bu, wr, ur, br, wh, uh, bh):
    wspec = pl.BlockSpec((K, K), lambda b: (0, 0))
    return pl.pallas_call(
        _gru_body,
        grid=(B,),
        in_specs=[
            pl.BlockSpec((1, K, K), lambda b: (b, 0, 0)),
            pl.BlockSpec((1, K, 1), lambda b: (b, 0, 0)),
            pl.BlockSpec((1, K, K), lambda b: (b, 0, 0)),
            wspec, wspec, wspec, wspec, wspec, wspec, wspec, wspec, wspec,
        ],
        out_specs=pl.BlockSpec((1, K, K), lambda b: (b, 0, 0)),
        out_shape=jax.ShapeDtypeStruct((B, K, K), jnp.float32),
    )(zg, tanh_sc, prev_q, wu, uu, bu, wr, ur, br, wh, uh, bh)


def kernel(prev_Z, prev_Q, mask, ht, W_map, b_map, W_upd, U_upd, bias_upd,
           W_rst, U_rst, bias_rst, W_htl, U_htl, bias_htl):
    scorer, inv = _scorer(ht, W_map.T, b_map.reshape(1, D))
    raw = _score(prev_Z, scorer, inv)
    ids, fidx, tanhs, ent, pol = _topk(raw)
    zg = _sc_gather(prev_Z.reshape(B * N, D), fidx.reshape(B * K))
    new_q = _gru(zg.reshape(B, K, D), tanhs.reshape(B, K, 1), prev_Q,
                 W_upd, U_upd, bias_upd, W_rst, U_rst, bias_rst,
                 W_htl, U_htl, bias_htl)
    return new_q, pol[:, 0], scorer, ent[:, 0], ids
